# baseline (speedup 1.0000x reference)
import functools

import jax
import jax.numpy as jnp
from jax import lax
from jax.experimental import pallas as pl
from jax.experimental.pallas import tpu as pltpu
from jax.experimental.pallas import tpu_sc as plsc

_LANES = 128


def _transpose_pad_kernel(embt_ref, out_ref):
    d = embt_ref.shape[0]
    et = lax.transpose(embt_ref[...], (1, 0))
    out_ref[...] = jnp.pad(et, ((0, 0), (0, _LANES - d)))


def _make_sc_gather(n_rows, d, n_per_w, nc):
    mesh = plsc.VectorSubcoreMesh(core_axis_name="c", subcore_axis_name="s")

    @functools.partial(
        pl.kernel,
        mesh=mesh,
        out_type=jax.ShapeDtypeStruct((n_rows, d), jnp.float32),
        scratch_types=[
            pltpu.VMEM((n_per_w,), jnp.int32),
            pltpu.VMEM((n_per_w, d), jnp.float32),
            pltpu.SemaphoreType.DMA,
        ],
    )
    def gather_kernel(emb_hbm, idx_hbm, out_hbm, idx_v, rows_v, sem):
        wid = lax.axis_index("s") * nc + lax.axis_index("c")
        base = wid * n_per_w
        pltpu.sync_copy(idx_hbm.at[pl.ds(base, n_per_w)], idx_v)
        pltpu.async_copy(emb_hbm.at[idx_v], rows_v, sem).wait()
        pltpu.sync_copy(rows_v, out_hbm.at[pl.ds(base, n_per_w)])

    return gather_kernel


def _make_h_kernel(s):
    def h_kernel(e_ref, w1_ref, b1_ref, h_ref):
        acc = jnp.dot(e_ref[0], w1_ref[0], preferred_element_type=jnp.float32)
        for i in range(1, s):
            acc += jnp.dot(e_ref[i], w1_ref[i],
                           preferred_element_type=jnp.float32)
        h_ref[...] = jnp.tanh(acc + b1_ref[...])

    return h_kernel


def _make_logits_kernel(b_sz):
    def logits_kernel(h_ref, w2t_ref, b2_ref, out_ref):
        acc = lax.dot_general(
            w2t_ref[...], h_ref[...],
            dimension_numbers=(((1,), (1,)), ((), ())),
            preferred_element_type=jnp.float32)
        out_ref[...] = acc + lax.dot_general(
            b2_ref[0], jnp.ones((1, b_sz), jnp.float32),
            dimension_numbers=(((0,), (0,)), ((), ())),
            preferred_element_type=jnp.float32)

    return logits_kernel


def kernel(x, emb, W1, b1, W2, b2):
    b_sz, s = x.shape
    v, d = emb.shape
    h_sz = W1.shape[1]
    n = b_sz * s

    info = plsc.get_sparse_core_info()
    nw = info.num_cores * info.num_subcores
    n_per_w = n // nw

    vc = 16384
    emb_p = pl.pallas_call(
        _transpose_pad_kernel,
        grid=(pl.cdiv(v, vc),),
        in_specs=[pl.BlockSpec((d, vc), lambda i: (0, i))],
        out_specs=pl.BlockSpec((vc, _LANES), lambda i: (i, 0)),
        out_shape=jax.ShapeDtypeStruct((v, _LANES), jnp.float32),
        compiler_params=pltpu.CompilerParams(
            dimension_semantics=("arbitrary",)),
    )(emb.T)

    idx = x.T.reshape(n)
    e = _make_sc_gather(n, _LANES, n_per_w, info.num_cores)(emb_p, idx)
    e3 = e.reshape(s, b_sz, _LANES)
    w1_p = jnp.pad(W1.reshape(s, d, h_sz), ((0, 0), (0, _LANES - d), (0, 0)))

    h = pl.pallas_call(
        _make_h_kernel(s),
        grid=(1,),
        in_specs=[
            pl.BlockSpec((s, b_sz, _LANES), lambda i: (0, 0, 0)),
            pl.BlockSpec((s, _LANES, h_sz), lambda i: (0, 0, 0)),
            pl.BlockSpec((1, h_sz), lambda i: (0, 0)),
        ],
        out_specs=pl.BlockSpec((b_sz, h_sz), lambda i: (0, 0)),
        out_shape=jax.ShapeDtypeStruct((b_sz, h_sz), jnp.float32),
        compiler_params=pltpu.CompilerParams(
            dimension_semantics=("arbitrary",)),
    )(e3, w1_p, b1.reshape(1, h_sz))

    vb = 5120
    nvb = pl.cdiv(v, vb)
    b2_blocks = jnp.pad(b2, (0, nvb * vb - v)).reshape(nvb, 1, vb)

    logits_t = pl.pallas_call(
        _make_logits_kernel(b_sz),
        grid=(nvb,),
        in_specs=[
            pl.BlockSpec((b_sz, h_sz), lambda i: (0, 0)),
            pl.BlockSpec((vb, h_sz), lambda i: (i, 0)),
            pl.BlockSpec((1, 1, vb), lambda i: (i, 0, 0)),
        ],
        out_specs=pl.BlockSpec((vb, b_sz), lambda i: (i, 0)),
        out_shape=jax.ShapeDtypeStruct((v, b_sz), jnp.float32),
        compiler_params=pltpu.CompilerParams(
            dimension_semantics=("arbitrary",)),
    )(h, W2.T, b2_blocks)
    return logits_t.T


# speedup vs baseline: 1.0081x; 1.0081x over previous
import functools

import jax
import jax.numpy as jnp
from jax import lax
from jax.experimental import pallas as pl
from jax.experimental.pallas import tpu as pltpu
from jax.experimental.pallas import tpu_sc as plsc

_LANES = 128


def _transpose_pad_kernel(embt_ref, out_ref):
    d = embt_ref.shape[0]
    et = lax.transpose(embt_ref[...], (1, 0))
    out_ref[...] = jnp.pad(et, ((0, 0), (0, _LANES - d)))


def _make_sc_gather(n_rows, d, n_per_w, nc):
    mesh = plsc.VectorSubcoreMesh(core_axis_name="c", subcore_axis_name="s")

    @functools.partial(
        pl.kernel,
        mesh=mesh,
        out_type=jax.ShapeDtypeStruct((n_rows, d), jnp.float32),
        scratch_types=[
            pltpu.VMEM((n_per_w,), jnp.int32),
            pltpu.VMEM((n_per_w, d), jnp.float32),
            pltpu.SemaphoreType.DMA,
        ],
    )
    def gather_kernel(emb_hbm, idx_hbm, out_hbm, idx_v, rows_v, sem):
        wid = lax.axis_index("s") * nc + lax.axis_index("c")
        base = wid * n_per_w
        pltpu.sync_copy(idx_hbm.at[pl.ds(base, n_per_w)], idx_v)
        pltpu.async_copy(emb_hbm.at[idx_v], rows_v, sem).wait()
        pltpu.sync_copy(rows_v, out_hbm.at[pl.ds(base, n_per_w)])

    return gather_kernel


def _make_mlp_kernel(s, b_sz):
    def mlp_kernel(e_ref, w1_ref, b1_ref, w2t_ref, b2_ref, out_ref, h_ref):
        @pl.when(pl.program_id(0) == 0)
        def _():
            acc = jnp.dot(e_ref[0], w1_ref[0],
                          preferred_element_type=jnp.float32)
            for i in range(1, s):
                acc += jnp.dot(e_ref[i], w1_ref[i],
                               preferred_element_type=jnp.float32)
            h_ref[...] = jnp.tanh(acc + b1_ref[...])

        acc = lax.dot_general(
            w2t_ref[...], h_ref[...],
            dimension_numbers=(((1,), (1,)), ((), ())),
            preferred_element_type=jnp.float32)
        out_ref[...] = acc + lax.dot_general(
            b2_ref[0], jnp.ones((1, b_sz), jnp.float32),
            dimension_numbers=(((0,), (0,)), ((), ())),
            preferred_element_type=jnp.float32)

    return mlp_kernel


def kernel(x, emb, W1, b1, W2, b2):
    b_sz, s = x.shape
    v, d = emb.shape
    h_sz = W1.shape[1]
    n = b_sz * s

    info = plsc.get_sparse_core_info()
    nw = info.num_cores * info.num_subcores
    n_per_w = n // nw

    vc = 16384
    emb_p = pl.pallas_call(
        _transpose_pad_kernel,
        grid=(pl.cdiv(v, vc),),
        in_specs=[pl.BlockSpec((d, vc), lambda i: (0, i))],
        out_specs=pl.BlockSpec((vc, _LANES), lambda i: (i, 0)),
        out_shape=jax.ShapeDtypeStruct((v, _LANES), jnp.float32),
        compiler_params=pltpu.CompilerParams(
            dimension_semantics=("arbitrary",)),
    )(emb.T)

    idx = x.T.reshape(n)
    e = _make_sc_gather(n, _LANES, n_per_w, info.num_cores)(emb_p, idx)
    e3 = e.reshape(s, b_sz, _LANES)
    w1_p = jnp.pad(W1.reshape(s, d, h_sz), ((0, 0), (0, _LANES - d), (0, 0)))

    vb = 4096
    nvb = pl.cdiv(v, vb)
    b2_blocks = jnp.pad(b2, (0, nvb * vb - v)).reshape(nvb, 1, vb)

    logits_t = pl.pallas_call(
        _make_mlp_kernel(s, b_sz),
        grid=(nvb,),
        in_specs=[
            pl.BlockSpec((s, b_sz, _LANES), lambda i: (0, 0, 0)),
            pl.BlockSpec((s, _LANES, h_sz), lambda i: (0, 0, 0)),
            pl.BlockSpec((1, h_sz), lambda i: (0, 0)),
            pl.BlockSpec((vb, h_sz), lambda i: (i, 0)),
            pl.BlockSpec((1, 1, vb), lambda i: (i, 0, 0)),
        ],
        out_specs=pl.BlockSpec((vb, b_sz), lambda i: (i, 0)),
        out_shape=jax.ShapeDtypeStruct((v, b_sz), jnp.float32),
        scratch_shapes=[pltpu.VMEM((b_sz, h_sz), jnp.float32)],
        compiler_params=pltpu.CompilerParams(
            dimension_semantics=("arbitrary",)),
    )(e3, w1_p, b1.reshape(1, h_sz), W2.T, b2_blocks)
    return logits_t.T
